# baseline (speedup 1.0000x reference)
import functools

import jax
import jax.numpy as jnp
from jax.experimental import pallas as pl
from jax.experimental.pallas import tpu as pltpu

LOG2E = 1.4426950408889634
LN2 = 0.6931471805599453
NEGM = -1.0e7
NEGBIG = -1e30


def _exp2_int(d):
    e = jnp.maximum(d.astype(jnp.int32) + 127, 0)
    return jax.lax.bitcast_convert_type(
        jax.lax.shift_left(e, 23), jnp.float32)


def _ctc_fwd_kernel(len_ref, i0_ref, i1_ref, ext_ref, skipm_ref, logits_ref,
                    out_ref, m_ref, r_ref, onehot_ref, em_ref, er_ref,
                    *, tc, nchunks, b, c, spad):
    pid = pl.program_id(0)

    @pl.when(pid == 0)
    def _init():
        lane = jax.lax.broadcasted_iota(jnp.int32, (b, spad), 1)
        m_ref[...] = jnp.where(lane == 0, 0.0, NEGM).astype(jnp.float32)
        r_ref[...] = jnp.ones((b, spad), jnp.float32)
        cidx = jax.lax.broadcasted_iota(jnp.int32, (b, c, spad), 1)
        onehot_ref[...] = jnp.where(ext_ref[...][:, None, :] == cidx,
                                    LOG2E, 0.0).astype(jnp.float32)

    x = logits_ref[...]
    mx_ = jnp.max(x, axis=2, keepdims=True)
    lse2 = (jnp.log(jnp.sum(jnp.exp(x - mx_), axis=2, keepdims=True))
            + mx_) * LOG2E
    for bb in range(b):
        v = (jax.lax.dot(x[bb], onehot_ref[bb],
                         preferred_element_type=jnp.float32) - lse2[bb])
        vf = jnp.floor(v)
        em_ref[bb, 0:tc, :] = vf
        er_ref[bb, 0:tc, :] = jnp.exp2(v - vf)

    skipm = skipm_ref[...]
    lens = len_ref[...]
    negm1 = jnp.full((b, 1), NEGM, jnp.float32)
    negm2 = jnp.full((b, 2), NEGM, jnp.float32)
    one1 = jnp.ones((b, 1), jnp.float32)
    one2 = jnp.ones((b, 2), jnp.float32)
    mant_mask = jnp.int32(0x007FFFFF)
    one_bits = jnp.int32(0x3F800000)

    def make_step(masked):
        def step(i, M, r, em, er):
            M1 = jnp.concatenate([negm1, M[:, :-1]], axis=1)
            r1 = jnp.concatenate([one1, r[:, :-1]], axis=1)
            M2 = jnp.concatenate([negm2, M[:, :-2]], axis=1) + skipm
            r2 = jnp.concatenate([one2, r[:, :-2]], axis=1)
            Ms = jnp.maximum(jnp.maximum(M, M1), M2)
            rs = (r * _exp2_int(M - Ms) + r1 * _exp2_int(M1 - Ms)
                  + r2 * _exp2_int(M2 - Ms))
            rn = rs * er
            bits = jax.lax.bitcast_convert_type(rn, jnp.int32)
            ee = jax.lax.shift_right_arithmetic(bits, 23) - 127
            Mn = (Ms + em) + ee.astype(jnp.float32)
            rm = jax.lax.bitcast_convert_type(
                jnp.bitwise_or(jnp.bitwise_and(bits, mant_mask), one_bits),
                jnp.float32)
            if masked:
                t = pid * tc + i
                act = t < lens
                return jnp.where(act, Mn, M), jnp.where(act, rm, r)
            return Mn, rm

        def step8(j, carry):
            M, r = carry
            base = 8 * j
            for k in range(8):
                em = em_ref[:, base + k, :]
                er = er_ref[:, base + k, :]
                M, r = step(base + k, M, r, em, er)
            return M, r
        return step8

    carry0 = (m_ref[...], r_ref[...])
    @pl.when(pid < nchunks // 2)
    def _loop_active():
        M, r = jax.lax.fori_loop(0, tc // 8, make_step(False), carry0)
        m_ref[...], r_ref[...] = M, r

    @pl.when(pid >= nchunks // 2)
    def _loop_masked():
        M, r = jax.lax.fori_loop(0, tc // 8, make_step(True), carry0)
        m_ref[...], r_ref[...] = M, r

    @pl.when(pid == nchunks - 1)
    def _final():
        al = m_ref[...] + jnp.log2(r_ref[...])
        sidx = jax.lax.broadcasted_iota(jnp.int32, (b, spad), 1)
        a1 = jnp.max(jnp.where(sidx == i1_ref[...], al, NEGBIG),
                     axis=1, keepdims=True)
        a0 = jnp.max(jnp.where(sidx == i0_ref[...], al, NEGBIG),
                     axis=1, keepdims=True)
        mm = jnp.maximum(a1, a0)
        out_ref[...] = -LN2 * (mm + jnp.log2(jnp.exp2(a1 - mm)
                                             + jnp.exp2(a0 - mm)))


def kernel(logits, targets, logits_lengths, targets_lengths):
    B, T, C = logits.shape
    L = targets.shape[1]
    S = 2 * L + 1
    SPAD = ((S + 127) // 128) * 128

    targets = targets.astype(jnp.int32)
    tl = targets_lengths.astype(jnp.int32)
    lens = logits_lengths.astype(jnp.int32)[:, None]

    ext = jnp.zeros((B, SPAD), jnp.int32).at[:, 1:S:2].set(targets)
    prev_t = jnp.concatenate(
        [jnp.full((B, 1), -1, jnp.int32), targets[:, :-1]], axis=1)
    skipm = jnp.full((B, SPAD), NEGM, jnp.float32).at[:, 1:S:2].set(
        jnp.where(targets != prev_t, 0.0, NEGM))
    i1 = (2 * tl)[:, None]
    i0 = (2 * tl - 1)[:, None]

    nchunks = 8
    tc = T // nchunks

    out = pl.pallas_call(
        functools.partial(_ctc_fwd_kernel, tc=tc, nchunks=nchunks,
                          b=B, c=C, spad=SPAD),
        grid=(nchunks,),
        in_specs=[
            pl.BlockSpec((B, 1), lambda i: (0, 0)),
            pl.BlockSpec((B, 1), lambda i: (0, 0)),
            pl.BlockSpec((B, 1), lambda i: (0, 0)),
            pl.BlockSpec((B, SPAD), lambda i: (0, 0)),
            pl.BlockSpec((B, SPAD), lambda i: (0, 0)),
            pl.BlockSpec((B, tc, C), lambda i: (0, i, 0)),
        ],
        out_specs=pl.BlockSpec((B, 1), lambda i: (0, 0)),
        out_shape=jax.ShapeDtypeStruct((B, 1), jnp.float32),
        scratch_shapes=[
            pltpu.VMEM((B, SPAD), jnp.float32),
            pltpu.VMEM((B, SPAD), jnp.float32),
            pltpu.VMEM((B, C, SPAD), jnp.float32),
            pltpu.VMEM((B, tc + 8, SPAD), jnp.float32),
            pltpu.VMEM((B, tc + 8, SPAD), jnp.float32),
        ],
        compiler_params=pltpu.CompilerParams(
            dimension_semantics=("arbitrary",)),
    )(lens, i0, i1, ext, skipm, logits)
    return out[:, 0]


# speedup vs baseline: 1.2796x; 1.2796x over previous
import functools

import jax
import jax.numpy as jnp
from jax.experimental import pallas as pl
from jax.experimental.pallas import tpu as pltpu

NEGBIG = -1e30


LOG2E = 1.4426950408889634
LN2 = 0.6931471805599453


def _ctc_fwd_kernel(len_ref, i0_ref, i1_ref, ext_ref, skipneg_ref, logits_ref,
                    out_ref, alpha_ref, onehot_ref, emit_ref,
                    *, tc, nchunks, b, c, spad):
    pid = pl.program_id(0)

    @pl.when(pid == 0)
    def _init():
        lane = jax.lax.broadcasted_iota(jnp.int32, (b, spad), 1)
        alpha_ref[...] = jnp.where(lane == 0, 0.0, NEGBIG).astype(jnp.float32)
        cidx = jax.lax.broadcasted_iota(jnp.int32, (b, c, spad), 1)
        onehot_ref[...] = jnp.where(ext_ref[...][:, None, :] == cidx,
                                    LOG2E, 0.0).astype(jnp.float32)

    x = logits_ref[...]
    m = jnp.max(x, axis=2, keepdims=True)
    lse2 = (jnp.log(jnp.sum(jnp.exp(x - m), axis=2, keepdims=True))
            + m) * LOG2E
    for bb in range(b):
        emit_ref[bb, 0:tc, :] = (jax.lax.dot(
            x[bb], onehot_ref[bb], preferred_element_type=jnp.float32)
            - lse2[bb])

    skipneg = skipneg_ref[...]
    lens = len_ref[...]
    negcol1 = jnp.full((b, 1), NEGBIG, jnp.float32)
    negcol2 = jnp.full((b, 2), NEGBIG, jnp.float32)

    def make_step(masked):
        def step(i, alpha, emit):
            p1 = jnp.concatenate([negcol1, alpha[:, :-1]], axis=1)
            p2 = jnp.concatenate([negcol2, alpha[:, :-2]], axis=1) + skipneg
            t1 = jnp.minimum(alpha, p1)
            t2 = jnp.maximum(alpha, p1)
            mx = jnp.maximum(t2, p2)
            mn = jnp.minimum(t1, p2)
            mid = jnp.maximum(t1, jnp.minimum(t2, p2))
            new = jnp.log2(1.0 + (jnp.exp2(mid - mx) + jnp.exp2(mn - mx))
                           ) + (mx + emit)
            if masked:
                t = pid * tc + i
                return jnp.where(t < lens, new, alpha)
            return new

        def step8(j, carry):
            alpha, e0 = carry
            base = 8 * j
            es = [e0] + [emit_ref[:, base + k, :] for k in range(1, 8)]
            for k in range(8):
                alpha = step(base + k, alpha, es[k])
            e_next = emit_ref[:, base + 8, :]
            return alpha, e_next
        return step8

    @pl.when(pid < nchunks // 2)
    def _loop_active():
        alpha_ref[...] = jax.lax.fori_loop(
            0, tc // 8, make_step(False),
            (alpha_ref[...], emit_ref[:, 0, :]))[0]

    @pl.when(pid >= nchunks // 2)
    def _loop_masked():
        alpha_ref[...] = jax.lax.fori_loop(
            0, tc // 8, make_step(True),
            (alpha_ref[...], emit_ref[:, 0, :]))[0]

    @pl.when(pid == nchunks - 1)
    def _final():
        alpha = alpha_ref[...]
        sidx = jax.lax.broadcasted_iota(jnp.int32, (b, spad), 1)
        a1 = jnp.max(jnp.where(sidx == i1_ref[...], alpha, NEGBIG),
                     axis=1, keepdims=True)
        a0 = jnp.max(jnp.where(sidx == i0_ref[...], alpha, NEGBIG),
                     axis=1, keepdims=True)
        mm = jnp.maximum(a1, a0)
        out_ref[...] = -LN2 * (mm + jnp.log2(jnp.exp2(a1 - mm)
                                             + jnp.exp2(a0 - mm)))


def kernel(logits, targets, logits_lengths, targets_lengths):
    B, T, C = logits.shape
    L = targets.shape[1]
    S = 2 * L + 1
    SPAD = ((S + 127) // 128) * 128

    targets = targets.astype(jnp.int32)
    tl = targets_lengths.astype(jnp.int32)
    lens = logits_lengths.astype(jnp.int32)[:, None]

    ext = jnp.zeros((B, SPAD), jnp.int32).at[:, 1:S:2].set(targets)
    prev_t = jnp.concatenate(
        [jnp.full((B, 1), -1, jnp.int32), targets[:, :-1]], axis=1)
    skipneg = jnp.full((B, SPAD), NEGBIG, jnp.float32).at[:, 1:S:2].set(
        jnp.where(targets != prev_t, 0.0, NEGBIG))
    i1 = (2 * tl)[:, None]
    i0 = (2 * tl - 1)[:, None]

    nchunks = 4
    tc = T // nchunks

    out = pl.pallas_call(
        functools.partial(_ctc_fwd_kernel, tc=tc, nchunks=nchunks,
                          b=B, c=C, spad=SPAD),
        grid=(nchunks,),
        in_specs=[
            pl.BlockSpec((B, 1), lambda i: (0, 0)),
            pl.BlockSpec((B, 1), lambda i: (0, 0)),
            pl.BlockSpec((B, 1), lambda i: (0, 0)),
            pl.BlockSpec((B, SPAD), lambda i: (0, 0)),
            pl.BlockSpec((B, SPAD), lambda i: (0, 0)),
            pl.BlockSpec((B, tc, C), lambda i: (0, i, 0)),
        ],
        out_specs=pl.BlockSpec((B, 1), lambda i: (0, 0)),
        out_shape=jax.ShapeDtypeStruct((B, 1), jnp.float32),
        scratch_shapes=[
            pltpu.VMEM((B, SPAD), jnp.float32),
            pltpu.VMEM((B, C, SPAD), jnp.float32),
            pltpu.VMEM((B, tc + 8, SPAD), jnp.float32),
        ],
        compiler_params=pltpu.CompilerParams(
            dimension_semantics=("arbitrary",)),
    )(lens, i0, i1, ext, skipneg, logits)
    return out[:, 0]
